# baseline (speedup 1.0000x reference)
import functools
import jax
import jax.numpy as jnp
from jax import lax
from jax.experimental import pallas as pl
from jax.experimental.pallas import tpu as pltpu
from jax.experimental.pallas import tpu_sc as plsc

_BLK = 4096
_S = 2


def _logits_body(*refs):
    xs = refs[:_S]
    w1 = refs[_S][...]
    b1 = refs[_S + 1][...]
    w2 = refs[_S + 2][...]
    b2col = refs[_S + 3][...].T
    out_ref = refs[_S + 4]
    for j in range(_S):
        x = xs[j][...]
        h = jnp.dot(x, w1, preferred_element_type=jnp.float32) + b1
        h = 0.5 * h * (1.0 + jax.lax.erf(h * 0.7071067811865476))
        lt = jax.lax.dot_general(
            w2, h, (((0,), (1,)), ((), ())),
            preferred_element_type=jnp.float32,
        ) + b2col
        out_ref[pl.ds(j * _BLK, _BLK), :] = lt.T


def _tc_logits(x, W1, b1, W2, b2):
    n, d_in = x.shape
    d_h = W1.shape[1]
    n_out = W2.shape[1]
    super_blk = _S * _BLK
    grid = (n // super_blk,)

    def mk_x_spec(j):
        return pl.BlockSpec((_BLK, d_in), lambda i, j=j: (i * _S + j, 0))

    return pl.pallas_call(
        _logits_body,
        grid=grid,
        in_specs=[mk_x_spec(j) for j in range(_S)] + [
            pl.BlockSpec((d_in, d_h), lambda i: (0, 0)),
            pl.BlockSpec((1, d_h), lambda i: (0, 0)),
            pl.BlockSpec((d_h, n_out), lambda i: (0, 0)),
            pl.BlockSpec((1, n_out), lambda i: (0, 0)),
        ],
        out_specs=[pl.BlockSpec((super_blk, n_out), lambda i: (i, 0))],
        out_shape=[jax.ShapeDtypeStruct((n, n_out), jnp.float32)],
        compiler_params=pltpu.CompilerParams(
            dimension_semantics=("arbitrary",),
        ),
    )(*([x] * _S), W1, b1.reshape(1, d_h), W2, b2.reshape(1, n_out))[0]


def _sc_gate(logits):
    n, n_out = logits.shape
    num_cores, num_subcores, lanes = 2, 16, 16
    nw = num_cores * num_subcores
    per_w = n // nw
    flat = per_w * n_out
    mesh = plsc.VectorSubcoreMesh(core_axis_name="c", subcore_axis_name="s")

    @functools.partial(
        pl.kernel,
        mesh=mesh,
        compiler_params=pltpu.CompilerParams(needs_layout_passes=False),
        out_type=[
            jax.ShapeDtypeStruct((n * n_out,), jnp.float32),
            jax.ShapeDtypeStruct((n * n_out,), jnp.float32),
        ],
        scratch_types=[
            pltpu.VMEM((flat,), jnp.float32),
            pltpu.VMEM((flat,), jnp.float32),
            pltpu.VMEM((flat,), jnp.float32),
        ],
    )
    def k(logits_hbm, gated_hbm, mask_hbm, lo_v, ga_v, ma_v):
        wid = lax.axis_index("s") * num_cores + lax.axis_index("c")
        base = wid * flat
        pltpu.sync_copy(logits_hbm.at[pl.ds(base, flat)], lo_v)
        def body(i, _):
            idx = (lax.iota(jnp.int32, lanes) + i * lanes) * n_out
            l0 = plsc.load_gather(lo_v, [idx])
            l1 = plsc.load_gather(lo_v, [idx + 1])
            l2 = plsc.load_gather(lo_v, [idx + 2])
            m = jnp.maximum(jnp.maximum(l0, l1), l2)
            e0 = jnp.exp(l0 - m)
            e1 = jnp.exp(l1 - m)
            e2 = jnp.exp(l2 - m)
            s = e0 + e1 + e2
            g0 = e0 / s
            g1 = e1 / s
            g2 = e2 / s
            excl2 = (g2 <= g0) & (g2 <= g1)
            excl1 = (~excl2) & (g1 <= g0) & (g1 < g2)
            excl0 = (~excl2) & (~excl1)
            one = jnp.ones((lanes,), jnp.float32)
            zero = jnp.zeros((lanes,), jnp.float32)
            m0 = jnp.where(excl0, zero, one)
            m1 = jnp.where(excl1, zero, one)
            m2 = jnp.where(excl2, zero, one)
            t0g = g0 * m0
            t1g = g1 * m1
            t2g = g2 * m2
            d = t0g + t1g + t2g + 1e-8
            plsc.store_scatter(ga_v, [idx], t0g / d)
            plsc.store_scatter(ga_v, [idx + 1], t1g / d)
            plsc.store_scatter(ga_v, [idx + 2], t2g / d)
            plsc.store_scatter(ma_v, [idx], m0)
            plsc.store_scatter(ma_v, [idx + 1], m1)
            plsc.store_scatter(ma_v, [idx + 2], m2)
            return 0
        lax.fori_loop(0, per_w // lanes, body, 0)
        pltpu.sync_copy(ga_v, gated_hbm.at[pl.ds(base, flat)])
        pltpu.sync_copy(ma_v, mask_hbm.at[pl.ds(base, flat)])

    gated_flat, mask_flat = k(logits.reshape(n * n_out))
    return gated_flat.reshape(n, n_out), mask_flat.reshape(n, n_out)


def kernel(combined_pooled_feat, W1, b1, W2, b2):
    logits = _tc_logits(combined_pooled_feat, W1, b1, W2, b2)
    return _sc_gate(logits)


# speedup vs baseline: 1.9806x; 1.9806x over previous
import jax
import jax.numpy as jnp
from jax.experimental import pallas as pl
from jax.experimental.pallas import tpu as pltpu

_BLK = 2048
_S = 4
_NEG = -1e30


def _gate_chunk(x, w1, b1, w2, b2col):
    h = jnp.dot(x, w1, preferred_element_type=jnp.float32) + b1
    h = 0.5 * h * (1.0 + jax.lax.erf(h * 0.7071067811865476))
    lt = jax.lax.dot_general(
        w2, h, (((0,), (1,)), ((), ())),
        preferred_element_type=jnp.float32,
    ) + b2col
    m = jnp.max(lt, axis=0, keepdims=True)
    e = jnp.exp(lt - m)
    s = jnp.sum(e, axis=0, keepdims=True)
    g = e / s
    g0 = g[0:1, :]
    g1 = g[1:2, :]
    g2 = g[2:3, :]
    excl2 = (g2 <= g0) & (g2 <= g1)
    excl1 = (~excl2) & (g1 <= g0) & (g1 < g2)
    excl0 = (~excl2) & (~excl1)
    ones = jnp.ones_like(g0)
    zeros = jnp.zeros_like(g0)
    mt = jnp.concatenate(
        [
            jnp.where(excl0, zeros, ones),
            jnp.where(excl1, zeros, ones),
            jnp.where(excl2, zeros, ones),
        ],
        axis=0,
    )
    gt = g[0:3, :] * mt
    gt = gt / (jnp.sum(gt, axis=0, keepdims=True) + 1e-8)
    return gt.T, mt.T


def _gate_body(*refs):
    xs = refs[:_S]
    w1 = refs[_S][...]
    b1 = refs[_S + 1][...]
    w2 = refs[_S + 2][...]
    b2col = refs[_S + 3][...].T
    gated_ref, mask_ref = refs[_S + 4], refs[_S + 5]
    for j in range(_S):
        gated, mask = _gate_chunk(xs[j][...], w1, b1, w2, b2col)
        gated_ref[pl.ds(j * _BLK, _BLK), :] = gated
        mask_ref[pl.ds(j * _BLK, _BLK), :] = mask


def kernel(combined_pooled_feat, W1, b1, W2, b2):
    n, d_in = combined_pooled_feat.shape
    d_h = W1.shape[1]
    n_out = W2.shape[1]
    super_blk = _S * _BLK
    grid = (n // super_blk,)

    def mk_x_spec(j):
        return pl.BlockSpec((_BLK, d_in), lambda i, j=j: (i * _S + j, 0))

    gated, mask = pl.pallas_call(
        _gate_body,
        grid=grid,
        in_specs=[mk_x_spec(j) for j in range(_S)] + [
            pl.BlockSpec((d_in, d_h), lambda i: (0, 0)),
            pl.BlockSpec((1, d_h), lambda i: (0, 0)),
            pl.BlockSpec((d_h, n_out), lambda i: (0, 0)),
            pl.BlockSpec((1, n_out), lambda i: (0, 0)),
        ],
        out_specs=[
            pl.BlockSpec((super_blk, n_out), lambda i: (i, 0)),
            pl.BlockSpec((super_blk, n_out), lambda i: (i, 0)),
        ],
        out_shape=[
            jax.ShapeDtypeStruct((n, n_out), jnp.float32),
            jax.ShapeDtypeStruct((n, n_out), jnp.float32),
        ],
        compiler_params=pltpu.CompilerParams(
            dimension_semantics=("arbitrary",),
        ),
    )(*([combined_pooled_feat] * _S), W1, b1.reshape(1, d_h), W2, b2.reshape(1, n_out))
    return (gated, mask)
